# baseline (speedup 1.0000x reference)
import functools

import jax
import jax.numpy as jnp
from jax import lax
from jax.experimental import pallas as pl
from jax.experimental.pallas import tpu as pltpu
from jax.experimental.pallas import tpu_sc as plsc

N = 10000
E = 320000
F = 64
NP = 10240
NC = 2
NS = 16
NW = NC * NS
EW = E // NW
CH = 80
NCH = EW // CH
NBUF = 4
RT = NP // NS

_mesh = plsc.VectorSubcoreMesh(core_axis_name="c", subcore_axis_name="s")
_sc_params = pltpu.CompilerParams(use_tc_tiling_on_sc=False)


@functools.partial(
    pl.kernel,
    out_type=(jax.ShapeDtypeStruct((NP,), jnp.float32),
              jax.ShapeDtypeStruct((NP,), jnp.float32)),
    mesh=_mesh,
    compiler_params=_sc_params,
    scratch_types=[
        pltpu.VMEM_SHARED((NP,), jnp.float32),
        pltpu.VMEM((RT,), jnp.float32),
        pltpu.VMEM((NCH, CH), jnp.int32),
        pltpu.VMEM((CH,), jnp.float32),
        pltpu.SemaphoreType.DMA,
    ],
)
def _deg_sc(dstp_hbm, out0_hbm, out1_hbm, acc, stage, didx, ones, isem):
    cid = lax.axis_index("c")
    sid = lax.axis_index("s")
    wid = sid * NC + cid

    c0 = wid * NCH
    pltpu.async_copy(dstp_hbm.at[pl.ds(c0, NCH)], didx, isem)

    z16 = jnp.zeros((16,), jnp.float32)
    o16 = jnp.ones((16,), jnp.float32)

    def zl(i, c):
        stage[pl.ds(i * 16, 16)] = z16
        return c

    lax.fori_loop(0, RT // 16, zl, 0)

    def ol(i, c):
        ones[pl.ds(i * 16, 16)] = o16
        return c

    lax.fori_loop(0, CH // 16, ol, 0)

    row0 = sid * RT
    pltpu.sync_copy(stage, acc.at[pl.ds(row0, RT)])
    pltpu.make_async_copy(dstp_hbm.at[pl.ds(c0, NCH)], didx, isem).wait()
    plsc.subcore_barrier()

    def chunk(i, c):
        pltpu.sync_copy(ones, acc.at[didx.at[i]], add=True)
        return c

    lax.fori_loop(0, NCH, chunk, 0)
    plsc.subcore_barrier()

    pltpu.sync_copy(acc.at[pl.ds(row0, RT)], stage)

    @pl.when(cid == 0)
    def _():
        pltpu.sync_copy(stage, out0_hbm.at[pl.ds(row0, RT)])

    @pl.when(cid == 1)
    def _():
        pltpu.sync_copy(stage, out1_hbm.at[pl.ds(row0, RT)])


@functools.partial(
    pl.kernel,
    out_type=(jax.ShapeDtypeStruct((NP, F), jnp.float32),
              jax.ShapeDtypeStruct((NP, F), jnp.float32)),
    mesh=_mesh,
    compiler_params=_sc_params,
    scratch_types=[
        pltpu.VMEM_SHARED((NP, F), jnp.float32),
        pltpu.VMEM_SHARED((NP, F), jnp.float32),
        pltpu.VMEM((NCH, CH), jnp.int32),
        pltpu.VMEM((NCH, CH), jnp.int32),
        [pltpu.VMEM((CH, F), jnp.float32) for _ in range(NBUF)],
        [pltpu.SemaphoreType.DMA for _ in range(NBUF)],
        [pltpu.SemaphoreType.DMA for _ in range(NBUF)],
        pltpu.SemaphoreType.DMA,
    ],
)
def _prop_sc(t_hbm, src_hbm, dstp_hbm, out0_hbm, out1_hbm, acc, tsh,
             sidx, didx, rows, gsem, ssem, isem):
    cid = lax.axis_index("c")
    sid = lax.axis_index("s")
    wid = sid * NC + cid

    c0 = wid * NCH
    pltpu.async_copy(src_hbm.at[pl.ds(c0, NCH)], sidx, isem)
    pltpu.async_copy(dstp_hbm.at[pl.ds(c0, NCH)], didx, isem)

    z16 = jnp.zeros((16,), jnp.float32)

    def zl(i, c):
        rows[0][i // (F // 16), pl.ds((i % (F // 16)) * 16, 16)] = z16
        return c

    lax.fori_loop(0, CH * (F // 16), zl, 0)

    row0 = sid * RT

    def zacc(j, c):
        pltpu.sync_copy(rows[0], acc.at[pl.ds(row0 + j * CH, CH)])
        return c

    lax.fori_loop(0, RT // CH, zacc, 0)

    def st(j, c):
        r = row0 + j * CH
        pltpu.sync_copy(t_hbm.at[pl.ds(r, CH)], rows[1])
        pltpu.sync_copy(rows[1], tsh.at[pl.ds(r, CH)])
        return c

    lax.fori_loop(0, RT // CH, st, 0)

    pltpu.make_async_copy(src_hbm.at[pl.ds(c0, NCH)], sidx, isem).wait()
    pltpu.make_async_copy(dstp_hbm.at[pl.ds(c0, NCH)], didx, isem).wait()
    plsc.subcore_barrier()

    for b in range(NBUF):
        pltpu.async_copy(tsh.at[sidx.at[b]], rows[b], gsem[b])

    def body(j, c):
        i0 = NBUF * j
        for b in range(NBUF):
            pltpu.make_async_copy(
                tsh.at[sidx.at[0]], rows[b], gsem[b]).wait()
            pltpu.async_copy(
                rows[b], acc.at[didx.at[i0 + b]], ssem[b], add=True)
        for b in range(NBUF):
            @pl.when(i0 + b + NBUF < NCH)
            def _(b=b):
                pltpu.make_async_copy(
                    rows[b], acc.at[didx.at[0]], ssem[b]).wait()
                pltpu.async_copy(
                    tsh.at[sidx.at[i0 + NBUF + b]], rows[b], gsem[b])
        return c

    lax.fori_loop(0, NCH // NBUF, body, 0)

    for i in range(NBUF * (NCH // NBUF), NCH):
        b = i % NBUF
        pltpu.make_async_copy(tsh.at[sidx.at[0]], rows[b], gsem[b]).wait()
        pltpu.async_copy(rows[b], acc.at[didx.at[i]], ssem[b], add=True)

    for b in range(NBUF):
        pltpu.make_async_copy(rows[b], acc.at[didx.at[0]], ssem[b]).wait()
    plsc.subcore_barrier()

    def wb(j, c):
        pltpu.sync_copy(acc.at[pl.ds(row0 + j * CH, CH)], rows[0])

        @pl.when(cid == 0)
        def _():
            pltpu.sync_copy(rows[0], out0_hbm.at[pl.ds(row0 + j * CH, CH)])

        @pl.when(cid == 1)
        def _():
            pltpu.sync_copy(rows[0], out1_hbm.at[pl.ds(row0 + j * CH, CH)])

        return c

    lax.fori_loop(0, RT // CH, wb, 0)


BR = 2048


def _prep_body(src_ref, dst_ref, out_ref):
    s = src_ref[...]
    d = dst_ref[...]
    out_ref[...] = jnp.where(s != d, d, jnp.int32(N))


def _comb1_body(d0_ref, d1_ref, x_ref, w_ref, t1_ref, nrm_ref):
    deg = d0_ref[...] + d1_ref[...] + 1.0
    nrm = lax.rsqrt(jnp.maximum(deg, 1.0)).reshape(BR, 1)
    nrm_ref[...] = nrm
    y = lax.dot_general(x_ref[...], w_ref[...], (((1,), (1,)), ((), ())),
                        preferred_element_type=jnp.float32)
    t1_ref[...] = y * nrm


def _comb2_body(a0_ref, a1_ref, t1_ref, nrm_ref, t2_ref):
    nrm = nrm_ref[...]
    t2_ref[...] = (a0_ref[...] + a1_ref[...] + t1_ref[...]) * (nrm * nrm)


def _final_body(a0_ref, a1_ref, t2_ref, nrm_ref, b_ref, o_ref):
    o_ref[...] = (a0_ref[...] + a1_ref[...] + t2_ref[...]) * nrm_ref[...] \
        + b_ref[...]


def kernel(features, edge_index, W, b):
    src = edge_index[0]
    dst = edge_index[1]

    dstp = pl.pallas_call(
        _prep_body,
        out_shape=jax.ShapeDtypeStruct((2500, 128), jnp.int32),
    )(src.reshape(2500, 128), dst.reshape(2500, 128))
    dstp2 = dstp.reshape(E // CH, CH)
    src2 = src.reshape(E // CH, CH)

    d0, d1 = _deg_sc(dstp2)

    xp = jnp.pad(features, ((0, NP - N), (0, 0)))

    t1, nrm = pl.pallas_call(
        _comb1_body,
        grid=(NP // BR,),
        in_specs=[
            pl.BlockSpec((BR,), lambda i: (i,)),
            pl.BlockSpec((BR,), lambda i: (i,)),
            pl.BlockSpec((BR, 128), lambda i: (i, 0)),
            pl.BlockSpec((F, 128), lambda i: (0, 0)),
        ],
        out_specs=[
            pl.BlockSpec((BR, F), lambda i: (i, 0)),
            pl.BlockSpec((BR, 1), lambda i: (i, 0)),
        ],
        out_shape=(jax.ShapeDtypeStruct((NP, F), jnp.float32),
                   jax.ShapeDtypeStruct((NP, 1), jnp.float32)),
    )(d0, d1, xp, W)

    a10, a11 = _prop_sc(t1, src2, dstp2)
    t2 = pl.pallas_call(
        _comb2_body,
        grid=(NP // BR,),
        in_specs=[
            pl.BlockSpec((BR, F), lambda i: (i, 0)),
            pl.BlockSpec((BR, F), lambda i: (i, 0)),
            pl.BlockSpec((BR, F), lambda i: (i, 0)),
            pl.BlockSpec((BR, 1), lambda i: (i, 0)),
        ],
        out_specs=pl.BlockSpec((BR, F), lambda i: (i, 0)),
        out_shape=jax.ShapeDtypeStruct((NP, F), jnp.float32),
    )(a10, a11, t1, nrm)

    a20, a21 = _prop_sc(t2, src2, dstp2)

    BO = 1000
    out = pl.pallas_call(
        _final_body,
        grid=(N // BO,),
        in_specs=[
            pl.BlockSpec((BO, F), lambda i: (i, 0)),
            pl.BlockSpec((BO, F), lambda i: (i, 0)),
            pl.BlockSpec((BO, F), lambda i: (i, 0)),
            pl.BlockSpec((BO, 1), lambda i: (i, 0)),
            pl.BlockSpec((1, F), lambda i: (0, 0)),
        ],
        out_specs=pl.BlockSpec((BO, F), lambda i: (i, 0)),
        out_shape=jax.ShapeDtypeStruct((N, F), jnp.float32),
    )(a20, a21, t2, nrm, b.reshape(1, F))

    return out


# speedup vs baseline: 1.0380x; 1.0380x over previous
import functools

import jax
import jax.numpy as jnp
from jax import lax
from jax.experimental import pallas as pl
from jax.experimental.pallas import tpu as pltpu
from jax.experimental.pallas import tpu_sc as plsc

N = 10000
E = 320000
F = 64
NP = 10240
NC = 2
NS = 16
NW = NC * NS
EW = E // NW
CH = 80
NCH = EW // CH
NBUF = 5
RT = NP // NS

_mesh = plsc.VectorSubcoreMesh(core_axis_name="c", subcore_axis_name="s")
_sc_params = pltpu.CompilerParams(use_tc_tiling_on_sc=False)


@functools.partial(
    pl.kernel,
    out_type=(jax.ShapeDtypeStruct((NP,), jnp.float32),
              jax.ShapeDtypeStruct((NP,), jnp.float32)),
    mesh=_mesh,
    compiler_params=_sc_params,
    scratch_types=[
        pltpu.VMEM_SHARED((NP,), jnp.float32),
        pltpu.VMEM((RT,), jnp.float32),
        pltpu.VMEM((NCH, CH), jnp.int32),
        pltpu.VMEM((CH,), jnp.float32),
        pltpu.SemaphoreType.DMA,
    ],
)
def _deg_sc(dstp_hbm, out0_hbm, out1_hbm, acc, stage, didx, ones, isem):
    cid = lax.axis_index("c")
    sid = lax.axis_index("s")
    wid = sid * NC + cid

    c0 = wid * NCH
    pltpu.async_copy(dstp_hbm.at[pl.ds(c0, NCH)], didx, isem)

    z16 = jnp.zeros((16,), jnp.float32)
    o16 = jnp.ones((16,), jnp.float32)

    def zl(i, c):
        stage[pl.ds(i * 16, 16)] = z16
        return c

    lax.fori_loop(0, RT // 16, zl, 0)

    def ol(i, c):
        ones[pl.ds(i * 16, 16)] = o16
        return c

    lax.fori_loop(0, CH // 16, ol, 0)

    row0 = sid * RT
    pltpu.sync_copy(stage, acc.at[pl.ds(row0, RT)])
    pltpu.make_async_copy(dstp_hbm.at[pl.ds(c0, NCH)], didx, isem).wait()
    plsc.subcore_barrier()

    def chunk(i, c):
        pltpu.sync_copy(ones, acc.at[didx.at[i]], add=True)
        return c

    lax.fori_loop(0, NCH, chunk, 0)
    plsc.subcore_barrier()

    pltpu.sync_copy(acc.at[pl.ds(row0, RT)], stage)

    @pl.when(cid == 0)
    def _():
        pltpu.sync_copy(stage, out0_hbm.at[pl.ds(row0, RT)])

    @pl.when(cid == 1)
    def _():
        pltpu.sync_copy(stage, out1_hbm.at[pl.ds(row0, RT)])


@functools.partial(
    pl.kernel,
    out_type=(jax.ShapeDtypeStruct((NP, F), jnp.float32),
              jax.ShapeDtypeStruct((NP, F), jnp.float32)),
    mesh=_mesh,
    compiler_params=_sc_params,
    scratch_types=[
        pltpu.VMEM_SHARED((NP, F), jnp.float32),
        pltpu.VMEM_SHARED((NP, F), jnp.float32),
        pltpu.VMEM((NCH, CH), jnp.int32),
        pltpu.VMEM((NCH, CH), jnp.int32),
        [pltpu.VMEM((CH, F), jnp.float32) for _ in range(NBUF)],
        [pltpu.SemaphoreType.DMA for _ in range(NBUF)],
        [pltpu.SemaphoreType.DMA for _ in range(NBUF)],
        pltpu.SemaphoreType.DMA,
    ],
)
def _prop_sc(t_hbm, src_hbm, dstp_hbm, out0_hbm, out1_hbm, acc, tsh,
             sidx, didx, rows, gsem, ssem, isem):
    cid = lax.axis_index("c")
    sid = lax.axis_index("s")
    wid = sid * NC + cid

    c0 = wid * NCH
    pltpu.async_copy(src_hbm.at[pl.ds(c0, NCH)], sidx, isem)
    pltpu.async_copy(dstp_hbm.at[pl.ds(c0, NCH)], didx, isem)

    z16 = jnp.zeros((16,), jnp.float32)

    def zl(i, c):
        rows[0][i // (F // 16), pl.ds((i % (F // 16)) * 16, 16)] = z16
        return c

    lax.fori_loop(0, CH * (F // 16), zl, 0)

    row0 = sid * RT

    def zacc(j, c):
        pltpu.sync_copy(rows[0], acc.at[pl.ds(row0 + j * CH, CH)])
        return c

    lax.fori_loop(0, RT // CH, zacc, 0)

    def st(j, c):
        r = row0 + j * CH
        pltpu.sync_copy(t_hbm.at[pl.ds(r, CH)], rows[1])
        pltpu.sync_copy(rows[1], tsh.at[pl.ds(r, CH)])
        return c

    lax.fori_loop(0, RT // CH, st, 0)

    pltpu.make_async_copy(src_hbm.at[pl.ds(c0, NCH)], sidx, isem).wait()
    pltpu.make_async_copy(dstp_hbm.at[pl.ds(c0, NCH)], didx, isem).wait()
    plsc.subcore_barrier()

    for b in range(NBUF):
        pltpu.async_copy(tsh.at[sidx.at[b]], rows[b], gsem[b])

    def body(j, c):
        i0 = NBUF * j
        for b in range(NBUF):
            pltpu.make_async_copy(
                tsh.at[sidx.at[0]], rows[b], gsem[b]).wait()
            pltpu.async_copy(
                rows[b], acc.at[didx.at[i0 + b]], ssem[b], add=True)
        for b in range(NBUF):
            @pl.when(i0 + b + NBUF < NCH)
            def _(b=b):
                pltpu.make_async_copy(
                    rows[b], acc.at[didx.at[0]], ssem[b]).wait()
                pltpu.async_copy(
                    tsh.at[sidx.at[i0 + NBUF + b]], rows[b], gsem[b])
        return c

    lax.fori_loop(0, NCH // NBUF, body, 0)

    for i in range(NBUF * (NCH // NBUF), NCH):
        b = i % NBUF
        pltpu.make_async_copy(tsh.at[sidx.at[0]], rows[b], gsem[b]).wait()
        pltpu.async_copy(rows[b], acc.at[didx.at[i]], ssem[b], add=True)

    for b in range(NBUF):
        pltpu.make_async_copy(rows[b], acc.at[didx.at[0]], ssem[b]).wait()
    plsc.subcore_barrier()

    def wb(j, c):
        pltpu.sync_copy(acc.at[pl.ds(row0 + j * CH, CH)], rows[0])

        @pl.when(cid == 0)
        def _():
            pltpu.sync_copy(rows[0], out0_hbm.at[pl.ds(row0 + j * CH, CH)])

        @pl.when(cid == 1)
        def _():
            pltpu.sync_copy(rows[0], out1_hbm.at[pl.ds(row0 + j * CH, CH)])

        return c

    lax.fori_loop(0, RT // CH, wb, 0)


BR = 2048


def _prep_body(ei_ref, src_ref, dstp_ref):
    s = ei_ref[0]
    d = ei_ref[1]
    src_ref[...] = s
    dstp_ref[...] = jnp.where(s != d, d, jnp.int32(N))


def _comb1_body(d0_ref, d1_ref, x_ref, w_ref, t1_ref, nrm_ref):
    deg = d0_ref[...] + d1_ref[...] + 1.0
    nrm = lax.rsqrt(jnp.maximum(deg, 1.0)).reshape(BR, 1)
    nrm_ref[...] = nrm
    y = lax.dot_general(x_ref[...], w_ref[...], (((1,), (1,)), ((), ())),
                        preferred_element_type=jnp.float32)
    t1_ref[...] = y * nrm


def _comb2_body(a0_ref, a1_ref, t1_ref, nrm_ref, t2_ref):
    nrm = nrm_ref[...]
    t2_ref[...] = (a0_ref[...] + a1_ref[...] + t1_ref[...]) * (nrm * nrm)


def _final_body(a0_ref, a1_ref, t2_ref, nrm_ref, b_ref, o_ref):
    o_ref[...] = (a0_ref[...] + a1_ref[...] + t2_ref[...]) * nrm_ref[...] \
        + b_ref[...]


def kernel(features, edge_index, W, b):
    src_lin, dstp_lin = pl.pallas_call(
        _prep_body,
        out_shape=(jax.ShapeDtypeStruct((E,), jnp.int32),
                   jax.ShapeDtypeStruct((E,), jnp.int32)),
    )(edge_index)
    dstp2 = dstp_lin.reshape(E // CH, CH)
    src2 = src_lin.reshape(E // CH, CH)

    d0, d1 = _deg_sc(dstp2)

    xp = jnp.pad(features, ((0, NP - N), (0, 0)))

    t1, nrm = pl.pallas_call(
        _comb1_body,
        grid=(NP // BR,),
        in_specs=[
            pl.BlockSpec((BR,), lambda i: (i,)),
            pl.BlockSpec((BR,), lambda i: (i,)),
            pl.BlockSpec((BR, 128), lambda i: (i, 0)),
            pl.BlockSpec((F, 128), lambda i: (0, 0)),
        ],
        out_specs=[
            pl.BlockSpec((BR, F), lambda i: (i, 0)),
            pl.BlockSpec((BR, 1), lambda i: (i, 0)),
        ],
        out_shape=(jax.ShapeDtypeStruct((NP, F), jnp.float32),
                   jax.ShapeDtypeStruct((NP, 1), jnp.float32)),
    )(d0, d1, xp, W)

    a10, a11 = _prop_sc(t1, src2, dstp2)
    t2 = pl.pallas_call(
        _comb2_body,
        grid=(NP // BR,),
        in_specs=[
            pl.BlockSpec((BR, F), lambda i: (i, 0)),
            pl.BlockSpec((BR, F), lambda i: (i, 0)),
            pl.BlockSpec((BR, F), lambda i: (i, 0)),
            pl.BlockSpec((BR, 1), lambda i: (i, 0)),
        ],
        out_specs=pl.BlockSpec((BR, F), lambda i: (i, 0)),
        out_shape=jax.ShapeDtypeStruct((NP, F), jnp.float32),
    )(a10, a11, t1, nrm)

    a20, a21 = _prop_sc(t2, src2, dstp2)

    BO = 1000
    out = pl.pallas_call(
        _final_body,
        grid=(N // BO,),
        in_specs=[
            pl.BlockSpec((BO, F), lambda i: (i, 0)),
            pl.BlockSpec((BO, F), lambda i: (i, 0)),
            pl.BlockSpec((BO, F), lambda i: (i, 0)),
            pl.BlockSpec((BO, 1), lambda i: (i, 0)),
            pl.BlockSpec((1, F), lambda i: (0, 0)),
        ],
        out_specs=pl.BlockSpec((BO, F), lambda i: (i, 0)),
        out_shape=jax.ShapeDtypeStruct((N, F), jnp.float32),
    )(a20, a21, t2, nrm, b.reshape(1, F))

    return out


# speedup vs baseline: 1.0984x; 1.0582x over previous
import functools

import jax
import jax.numpy as jnp
from jax import lax
from jax.experimental import pallas as pl
from jax.experimental.pallas import tpu as pltpu
from jax.experimental.pallas import tpu_sc as plsc

N = 10000
E = 320000
F = 64
NP = 10240
NC = 2
NS = 16
NW = NC * NS
EW = E // NW
CH = 80
NCH = EW // CH
NBUF = 5
RT = NP // NS

_mesh = plsc.VectorSubcoreMesh(core_axis_name="c", subcore_axis_name="s")
_sc_params = pltpu.CompilerParams(use_tc_tiling_on_sc=False)


@functools.partial(
    pl.kernel,
    out_type=(jax.ShapeDtypeStruct((NP,), jnp.float32),
              jax.ShapeDtypeStruct((NP,), jnp.float32)),
    mesh=_mesh,
    compiler_params=_sc_params,
    scratch_types=[
        pltpu.VMEM_SHARED((NP,), jnp.float32),
        pltpu.VMEM((RT,), jnp.float32),
        pltpu.VMEM((NCH, CH), jnp.int32),
        pltpu.VMEM((CH,), jnp.float32),
        pltpu.SemaphoreType.DMA,
    ],
)
def _deg_sc(dstp_hbm, out0_hbm, out1_hbm, acc, stage, didx, ones, isem):
    cid = lax.axis_index("c")
    sid = lax.axis_index("s")
    wid = sid * NC + cid

    c0 = wid * NCH
    pltpu.async_copy(dstp_hbm.at[pl.ds(c0, NCH)], didx, isem)

    z16 = jnp.zeros((16,), jnp.float32)
    o16 = jnp.ones((16,), jnp.float32)

    def zl(i, c):
        stage[pl.ds(i * 16, 16)] = z16
        return c

    lax.fori_loop(0, RT // 16, zl, 0)

    def ol(i, c):
        ones[pl.ds(i * 16, 16)] = o16
        return c

    lax.fori_loop(0, CH // 16, ol, 0)

    row0 = sid * RT
    pltpu.sync_copy(stage, acc.at[pl.ds(row0, RT)])
    pltpu.make_async_copy(dstp_hbm.at[pl.ds(c0, NCH)], didx, isem).wait()
    plsc.subcore_barrier()

    def chunk(i, c):
        pltpu.sync_copy(ones, acc.at[didx.at[i]], add=True)
        return c

    lax.fori_loop(0, NCH, chunk, 0)
    plsc.subcore_barrier()

    pltpu.sync_copy(acc.at[pl.ds(row0, RT)], stage)

    @pl.when(cid == 0)
    def _():
        pltpu.sync_copy(stage, out0_hbm.at[pl.ds(row0, RT)])

    @pl.when(cid == 1)
    def _():
        pltpu.sync_copy(stage, out1_hbm.at[pl.ds(row0, RT)])


@functools.partial(
    pl.kernel,
    out_type=(jax.ShapeDtypeStruct((NP, F), jnp.float32),
              jax.ShapeDtypeStruct((NP, F), jnp.float32)),
    mesh=_mesh,
    compiler_params=_sc_params,
    scratch_types=[
        pltpu.VMEM_SHARED((NP, F), jnp.float32),
        pltpu.VMEM_SHARED((NP, F), jnp.float32),
        pltpu.VMEM((NCH, CH), jnp.int32),
        pltpu.VMEM((NCH, CH), jnp.int32),
        [pltpu.VMEM((CH, F), jnp.float32) for _ in range(NBUF)],
        [pltpu.SemaphoreType.DMA for _ in range(NBUF)],
        [pltpu.SemaphoreType.DMA for _ in range(NBUF)],
        pltpu.SemaphoreType.DMA,
    ],
)
def _prop_sc(t_hbm, src_hbm, dstp_hbm, out0_hbm, out1_hbm, acc, tsh,
             sidx, didx, rows, gsem, ssem, isem):
    cid = lax.axis_index("c")
    sid = lax.axis_index("s")
    wid = sid * NC + cid

    c0 = wid * NCH
    pltpu.async_copy(src_hbm.at[pl.ds(c0, NCH)], sidx, isem)
    pltpu.async_copy(dstp_hbm.at[pl.ds(c0, NCH)], didx, isem)

    z16 = jnp.zeros((16,), jnp.float32)

    def zl(i, c):
        rows[0][i // (F // 16), pl.ds((i % (F // 16)) * 16, 16)] = z16
        return c

    lax.fori_loop(0, CH * (F // 16), zl, 0)

    row0 = sid * RT

    def zacc(j, c):
        pltpu.sync_copy(rows[0], acc.at[pl.ds(row0 + j * CH, CH)])
        return c

    lax.fori_loop(0, RT // CH, zacc, 0)

    def st(j, c):
        r = row0 + j * CH
        pltpu.sync_copy(t_hbm.at[pl.ds(r, CH)], rows[1])
        pltpu.sync_copy(rows[1], tsh.at[pl.ds(r, CH)])
        return c

    lax.fori_loop(0, RT // CH, st, 0)

    pltpu.make_async_copy(src_hbm.at[pl.ds(c0, NCH)], sidx, isem).wait()
    pltpu.make_async_copy(dstp_hbm.at[pl.ds(c0, NCH)], didx, isem).wait()
    plsc.subcore_barrier()

    def gsrc(b):
        return t_hbm if b < 2 else tsh

    for b in range(NBUF):
        pltpu.async_copy(gsrc(b).at[sidx.at[b]], rows[b], gsem[b])

    def body(j, c):
        i0 = NBUF * j
        for b in range(NBUF):
            pltpu.make_async_copy(
                gsrc(b).at[sidx.at[0]], rows[b], gsem[b]).wait()
            pltpu.async_copy(
                rows[b], acc.at[didx.at[i0 + b]], ssem[b], add=True)
        for b in range(NBUF):
            @pl.when(i0 + b + NBUF < NCH)
            def _(b=b):
                pltpu.make_async_copy(
                    rows[b], acc.at[didx.at[0]], ssem[b]).wait()
                pltpu.async_copy(
                    gsrc(b).at[sidx.at[i0 + NBUF + b]], rows[b], gsem[b])
        return c

    lax.fori_loop(0, NCH // NBUF, body, 0)

    for i in range(NBUF * (NCH // NBUF), NCH):
        b = i % NBUF
        pltpu.make_async_copy(tsh.at[sidx.at[0]], rows[b], gsem[b]).wait()
        pltpu.async_copy(rows[b], acc.at[didx.at[i]], ssem[b], add=True)

    for b in range(NBUF):
        pltpu.make_async_copy(rows[b], acc.at[didx.at[0]], ssem[b]).wait()
    plsc.subcore_barrier()

    def wb(j, c):
        pltpu.sync_copy(acc.at[pl.ds(row0 + j * CH, CH)], rows[0])

        @pl.when(cid == 0)
        def _():
            pltpu.sync_copy(rows[0], out0_hbm.at[pl.ds(row0 + j * CH, CH)])

        @pl.when(cid == 1)
        def _():
            pltpu.sync_copy(rows[0], out1_hbm.at[pl.ds(row0 + j * CH, CH)])

        return c

    lax.fori_loop(0, RT // CH, wb, 0)


BR = 2048


def _prep_body(ei_ref, src_ref, dstp_ref):
    s = ei_ref[0]
    d = ei_ref[1]
    src_ref[...] = s
    dstp_ref[...] = jnp.where(s != d, d, jnp.int32(N))


def _comb1_body(d0_ref, d1_ref, x_ref, w_ref, t1_ref, nrm_ref):
    deg = d0_ref[...] + d1_ref[...] + 1.0
    nrm = lax.rsqrt(jnp.maximum(deg, 1.0)).reshape(BR, 1)
    nrm_ref[...] = nrm
    y = lax.dot_general(x_ref[...], w_ref[...], (((1,), (1,)), ((), ())),
                        preferred_element_type=jnp.float32)
    t1_ref[...] = y * nrm


def _comb2_body(a0_ref, a1_ref, t1_ref, nrm_ref, t2_ref):
    nrm = nrm_ref[...]
    t2_ref[...] = (a0_ref[...] + a1_ref[...] + t1_ref[...]) * (nrm * nrm)


def _final_body(a0_ref, a1_ref, t2_ref, nrm_ref, b_ref, o_ref):
    o_ref[...] = (a0_ref[...] + a1_ref[...] + t2_ref[...]) * nrm_ref[...] \
        + b_ref[...]


def kernel(features, edge_index, W, b):
    src_lin, dstp_lin = pl.pallas_call(
        _prep_body,
        out_shape=(jax.ShapeDtypeStruct((E,), jnp.int32),
                   jax.ShapeDtypeStruct((E,), jnp.int32)),
    )(edge_index)
    dstp2 = dstp_lin.reshape(E // CH, CH)
    src2 = src_lin.reshape(E // CH, CH)

    d0, d1 = _deg_sc(dstp2)

    xp = jnp.pad(features, ((0, NP - N), (0, 0)))

    t1, nrm = pl.pallas_call(
        _comb1_body,
        grid=(NP // BR,),
        in_specs=[
            pl.BlockSpec((BR,), lambda i: (i,)),
            pl.BlockSpec((BR,), lambda i: (i,)),
            pl.BlockSpec((BR, 128), lambda i: (i, 0)),
            pl.BlockSpec((F, 128), lambda i: (0, 0)),
        ],
        out_specs=[
            pl.BlockSpec((BR, F), lambda i: (i, 0)),
            pl.BlockSpec((BR, 1), lambda i: (i, 0)),
        ],
        out_shape=(jax.ShapeDtypeStruct((NP, F), jnp.float32),
                   jax.ShapeDtypeStruct((NP, 1), jnp.float32)),
    )(d0, d1, xp, W)

    a10, a11 = _prop_sc(t1, src2, dstp2)
    t2 = pl.pallas_call(
        _comb2_body,
        grid=(NP // BR,),
        in_specs=[
            pl.BlockSpec((BR, F), lambda i: (i, 0)),
            pl.BlockSpec((BR, F), lambda i: (i, 0)),
            pl.BlockSpec((BR, F), lambda i: (i, 0)),
            pl.BlockSpec((BR, 1), lambda i: (i, 0)),
        ],
        out_specs=pl.BlockSpec((BR, F), lambda i: (i, 0)),
        out_shape=jax.ShapeDtypeStruct((NP, F), jnp.float32),
    )(a10, a11, t1, nrm)

    a20, a21 = _prop_sc(t2, src2, dstp2)

    BO = 1000
    out = pl.pallas_call(
        _final_body,
        grid=(N // BO,),
        in_specs=[
            pl.BlockSpec((BO, F), lambda i: (i, 0)),
            pl.BlockSpec((BO, F), lambda i: (i, 0)),
            pl.BlockSpec((BO, F), lambda i: (i, 0)),
            pl.BlockSpec((BO, 1), lambda i: (i, 0)),
            pl.BlockSpec((1, F), lambda i: (0, 0)),
        ],
        out_specs=pl.BlockSpec((BO, F), lambda i: (i, 0)),
        out_shape=jax.ShapeDtypeStruct((N, F), jnp.float32),
    )(a20, a21, t2, nrm, b.reshape(1, F))

    return out
